# baseline (speedup 1.0000x reference)
import functools

import jax
import jax.numpy as jnp
from jax import lax
from jax.experimental import pallas as pl
from jax.experimental.pallas import tpu as pltpu
from jax.experimental.pallas import tpu_sc as plsc

N = 16384
K = 8192
D = 256
BETA = 0.5
EPS = 1e-12

BM = 1024
BK = 8192
NRB = N // BM
NKB = K // BK

SC_CHUNK = 256


def _simil_body(z_ref, w_ref, idx_ref, loss_ref, wn_ref, st_ref, acc_s):
    i = pl.program_id(0)

    @pl.when(i == 0)
    def _wn():
        w = w_ref[...]
        s2 = jnp.sum(w * w, axis=1, keepdims=True)
        wn_ref[...] = w / jnp.maximum(jnp.sqrt(s2), EPS)

    z = z_ref[...]
    sz = jnp.sum(z * z, axis=1, keepdims=True)
    zn = z / jnp.maximum(jnp.sqrt(sz), EPS)
    st_ref[...] = lax.dot_general(
        wn_ref[...], zn,
        dimension_numbers=(((1,), (1,)), ((), ())),
        preferred_element_type=jnp.float32,
    )
    av = jnp.full((8, BM), -3.0, jnp.float32)
    ac = jnp.zeros((8, BM), jnp.int32)
    for v in range(K // 8):
        sv = st_ref[v * 8:(v + 1) * 8, :]
        b = sv > av
        ac = jnp.where(b, v, ac)
        av = jnp.where(b, sv, av)
    rows = lax.broadcasted_iota(jnp.int32, (8, BM), 0)
    g = ac * 8 + rows
    m = jnp.max(av, axis=0, keepdims=True)
    la = jnp.min(jnp.where(av == m, g, K), axis=0, keepdims=True)
    idx_ref[...] = la.reshape(1, 1, BM)
    part = jnp.sum(2.0 - 2.0 * m)
    prev = jnp.where(i == 0, 0.0, acc_s[0])
    acc_s[0] = prev + part

    @pl.when(i == NRB - 1)
    def _loss():
        loss_ref[0, 0] = (BETA + 1.0) * acc_s[0] / (N * D)


def _simil(z, W):
    return pl.pallas_call(
        _simil_body,
        grid=(NRB,),
        in_specs=[
            pl.BlockSpec((BM, D), lambda i: (i, 0)),
            pl.BlockSpec((K, D), lambda i: (0, 0)),
        ],
        out_specs=[
            pl.BlockSpec((1, 1, BM), lambda i: (i, 0, 0)),
            pl.BlockSpec((1, 1), lambda i: (0, 0), memory_space=pltpu.SMEM),
            pl.BlockSpec((K, D), lambda i: (0, 0)),
        ],
        out_shape=[
            jax.ShapeDtypeStruct((NRB, 1, BM), jnp.int32),
            jax.ShapeDtypeStruct((1, 1), jnp.float32),
            jax.ShapeDtypeStruct((K, D), jnp.float32),
        ],
        scratch_shapes=[
            pltpu.VMEM((K, BM), jnp.float32),
            pltpu.SMEM((1,), jnp.float32),
        ],
    )(z, W)


def _gather(Wn, idx):
    info = plsc.get_sparse_core_info()
    nw = info.num_cores * info.num_subcores
    b_per_w = N // nw
    nchunk = b_per_w // SC_CHUNK
    mesh = plsc.VectorSubcoreMesh(core_axis_name="c", subcore_axis_name="s")

    @functools.partial(
        pl.kernel,
        mesh=mesh,
        out_type=jax.ShapeDtypeStruct((N, D), jnp.float32),
        scratch_types=[
            pltpu.VMEM((SC_CHUNK,), jnp.int32),
            pltpu.VMEM((SC_CHUNK, D), jnp.float32),
            pltpu.SemaphoreType.DMA,
        ],
    )
    def k(table_hbm, idx_hbm, out_hbm, idx_v, rows_v, sem):
        wid = lax.axis_index("s") * info.num_cores + lax.axis_index("c")
        for c in range(nchunk):
            base = wid * b_per_w + c * SC_CHUNK
            pltpu.sync_copy(idx_hbm.at[pl.ds(base, SC_CHUNK)], idx_v)
            pltpu.async_copy(table_hbm.at[idx_v], rows_v, sem).wait()
            pltpu.sync_copy(rows_v, out_hbm.at[pl.ds(base, SC_CHUNK)])

    return k(Wn, idx)


def kernel(z, W):
    idx3d, loss2d, Wn = _simil(z, W)
    idx = idx3d.reshape(N)
    zq = _gather(Wn, idx)
    return (zq, idx, loss2d.reshape(()))
